# baseline (speedup 1.0000x reference)
import functools

import jax
import jax.numpy as jnp
from jax import lax
from jax.experimental import pallas as pl
from jax.experimental.pallas import tpu as pltpu
from jax.experimental.pallas import tpu_sc as plsc

N = 10000
E = 320000
F_IN = 24
H = 128
G = 64

NT = 32
NPT = 320
NPAD = NT * NPT
RB = NPAD // 4

CH = 256
PADM = 2 * CH
FLUSH = 2048
STG = FLUSH + PADM + 32
TRASH = STG - 16
ECAP = E + 2 * FLUSH
NEG = -1.0e30

_mesh = plsc.VectorSubcoreMesh(core_axis_name="c", subcore_axis_name="s")
_sc_params = pltpu.CompilerParams(needs_layout_passes=False)


def _wid():
    return lax.axis_index("s") * 2 + lax.axis_index("c")


@functools.partial(
    pl.kernel,
    mesh=_mesh,
    compiler_params=_sc_params,
    out_type=[
        jax.ShapeDtypeStruct((NT * ECAP,), jnp.int32),
        jax.ShapeDtypeStruct((NT * ECAP,), jnp.int32),
        jax.ShapeDtypeStruct((NT * 16,), jnp.int32),
    ],
    scratch_types=[
        pltpu.VMEM((4000,), jnp.int32),
        pltpu.VMEM((4000,), jnp.int32),
        pltpu.VMEM((STG,), jnp.int32),
        pltpu.VMEM((STG,), jnp.int32),
        pltpu.VMEM((16,), jnp.int32),
    ],
)
def _sc_filter(src_hbm, dst_hbm, srcl_hbm, dstl_hbm, cnt_hbm,
               dchunk, schunk, stg_s, stg_d, cvec):
    wid = _wid()
    base = wid * NPT
    lbase = pl.multiple_of(wid * ECAP, 8)
    lanes = lax.iota(jnp.int32, 16)

    def inner(j, carry):
        off, hoff = carry
        dv = dchunk[pl.ds(j * 16, 16)]
        sv = schunk[pl.ds(j * 16, 16)]
        dl = dv - base
        msk = (dl >= 0) & (dl < NPT)
        mi = jnp.where(msk, 1, 0)
        incl = plsc.cumsum(mi)
        rank = incl - mi
        pos = jnp.where(msk, off + rank, TRASH + lanes)
        plsc.store_scatter(stg_s, [pos], sv)
        plsc.store_scatter(stg_d, [pos], dl)
        off = off + incl[15]

        def do_flush(o, h):
            hb = pl.multiple_of(lbase + h, 8)
            pltpu.sync_copy(stg_s.at[pl.ds(0, FLUSH)],
                            srcl_hbm.at[pl.ds(hb, FLUSH)])
            pltpu.sync_copy(stg_d.at[pl.ds(0, FLUSH)],
                            dstl_hbm.at[pl.ds(hb, FLUSH)])
            stg_s[pl.ds(0, 16)] = stg_s[pl.ds(FLUSH, 16)]
            stg_d[pl.ds(0, 16)] = stg_d[pl.ds(FLUSH, 16)]
            return o - FLUSH, h + FLUSH

        return lax.cond(off >= FLUSH, do_flush, lambda o, h: (o, h), off, hoff)

    def chunk_body(ci, carry):
        pltpu.sync_copy(src_hbm.at[pl.ds(ci * 4000, 4000)], schunk)
        pltpu.sync_copy(dst_hbm.at[pl.ds(ci * 4000, 4000)], dchunk)
        return lax.fori_loop(0, 250, inner, carry)

    off, hoff = lax.fori_loop(0, E // 4000, chunk_body, (0, 0))

    a0 = (off // 16) * 16
    rem = off - a0
    keep_s = stg_s[pl.ds(a0, 16)]
    keep_d = stg_d[pl.ds(a0, 16)]
    stg_s[pl.ds(a0, 16)] = jnp.where(lanes < rem, keep_s, 0)
    stg_d[pl.ds(a0, 16)] = jnp.where(lanes < rem, keep_d, NPT)
    for k in range(1, PADM // 16):
        stg_s[pl.ds(a0 + 16 * k, 16)] = jnp.zeros((16,), jnp.int32)
        stg_d[pl.ds(a0 + 16 * k, 16)] = jnp.full((16,), NPT, jnp.int32)
    hb = pl.multiple_of(lbase + hoff, 8)
    pltpu.sync_copy(stg_s.at[pl.ds(0, FLUSH)], srcl_hbm.at[pl.ds(hb, FLUSH)])
    pltpu.sync_copy(stg_d.at[pl.ds(0, FLUSH)], dstl_hbm.at[pl.ds(hb, FLUSH)])

    total = hoff + ((off + PADM - 1) // PADM) * PADM
    cvec[...] = jnp.broadcast_to(total, (16,)).astype(jnp.int32)
    pltpu.sync_copy(cvec, cnt_hbm.at[pl.ds(pl.multiple_of(wid * 16, 8), 16)])


@functools.partial(
    pl.kernel,
    mesh=_mesh,
    compiler_params=_sc_params,
    out_type=jax.ShapeDtypeStruct((NPAD, H), jnp.float32),
    scratch_types=[
        pltpu.VMEM((NPT + 1, H), jnp.float32),
        pltpu.VMEM((2, CH), jnp.int32),
        pltpu.VMEM((2, CH, H), jnp.float32),
        pltpu.VMEM((2, CH), jnp.int32),
        pltpu.VMEM((16,), jnp.int32),
        pltpu.SemaphoreType.DMA,
        pltpu.SemaphoreType.DMA,
    ],
)
def _sc_segmax(q_hbm, srcl_hbm, dstl_hbm, cnt_hbm, s_hbm,
               s_loc, idx_v, rows_v, dl_vmem, cnt_vmem, sem0, sem1):
    wid = _wid()
    base = pl.multiple_of(wid * NPT, 8)
    lbase = pl.multiple_of(wid * ECAP, 8)
    pltpu.sync_copy(cnt_hbm.at[pl.ds(pl.multiple_of(wid * 16, 8), 16)], cnt_vmem)
    n = cnt_vmem[...][0]
    ng = n // CH

    negv = jnp.full((16,), NEG, jnp.float32)

    def initb(i, _):
        for c in range(H // 16):
            s_loc[i, pl.ds(c * 16, 16)] = negv
        return 0

    lax.fori_loop(0, NPT + 1, initb, 0)

    def start(slot, sem, g):
        gb = pl.multiple_of(lbase + g * CH, 8)
        pltpu.sync_copy(srcl_hbm.at[pl.ds(gb, CH)], idx_v.at[slot])
        pltpu.async_copy(q_hbm.at[idx_v.at[slot, pl.ds(0, 128)]],
                         rows_v.at[slot, pl.ds(0, 128)], sem)
        pltpu.async_copy(q_hbm.at[idx_v.at[slot, pl.ds(128, 128)]],
                         rows_v.at[slot, pl.ds(128, 128)], sem)
        pltpu.async_copy(dstl_hbm.at[pl.ds(gb, CH)], dl_vmem.at[slot], sem)

    def wait(slot, sem):
        pltpu.make_async_copy(q_hbm.at[pl.ds(0, CH)], rows_v.at[slot], sem).wait()
        pltpu.make_async_copy(dstl_hbm.at[pl.ds(0, CH)], dl_vmem.at[slot],
                              sem).wait()

    def drain(slot, g):
        def group_body(gi, _):
            dl16 = dl_vmem[slot, pl.ds(gi * 16, 16)]
            for t in range(16):
                d = dl16[t]
                e = gi * 16 + t
                for c in range(H // 16):
                    sl = pl.ds(c * 16, 16)
                    s_loc[d, sl] = jnp.maximum(s_loc[d, sl],
                                               rows_v[slot, e, sl])
            return 0

        lax.fori_loop(0, CH // 16, group_body, 0)

    @pl.when(ng > 0)
    def _():
        start(0, sem0, 0)

    def body(i, _):
        g0 = 2 * i
        start(1, sem1, g0 + 1)
        wait(0, sem0)
        drain(0, g0)

        @pl.when(g0 + 2 < ng)
        def _():
            start(0, sem0, g0 + 2)

        wait(1, sem1)
        drain(1, g0 + 1)
        return 0

    lax.fori_loop(0, ng // 2, body, 0)
    pltpu.sync_copy(s_loc.at[pl.ds(0, NPT)], s_hbm.at[pl.ds(base, NPT)])


def _mm_in_body(x_ref, w_ref, p_ref, q_ref):
    pq = jnp.dot(x_ref[...], w_ref[...], preferred_element_type=jnp.float32, precision=lax.Precision.HIGHEST)
    p_ref[...] = pq[:, :H]
    q_ref[...] = pq[:, H:]


def _mm_in(xp, wc):
    return pl.pallas_call(
        _mm_in_body,
        grid=(4,),
        in_specs=[pl.BlockSpec((RB, F_IN), lambda i: (i, 0)),
                  pl.BlockSpec((F_IN, 2 * H), lambda i: (0, 0))],
        out_specs=[pl.BlockSpec((RB, H), lambda i: (i, 0)),
                   pl.BlockSpec((RB, H), lambda i: (i, 0))],
        out_shape=[jax.ShapeDtypeStruct((NPAD, H), jnp.float32)] * 2,
    )(xp, wc)


def _stats_body(p_ref, b_ref, s_ref, h_ref, sums_ref):
    i = pl.program_id(0)
    h = jnp.maximum(p_ref[...] + b_ref[...] + s_ref[...], 0.0)
    h_ref[...] = h
    ps = jnp.sum(h, 0, keepdims=True)

    @pl.when(i == 0)
    def _():
        sums_ref[...] = ps

    @pl.when(i > 0)
    def _():
        sums_ref[...] = sums_ref[...] + ps


def _stats(p, b, s):
    return pl.pallas_call(
        _stats_body,
        grid=(4,),
        in_specs=[pl.BlockSpec((RB, H), lambda i: (i, 0)),
                  pl.BlockSpec((1, H), lambda i: (0, 0)),
                  pl.BlockSpec((RB, H), lambda i: (i, 0))],
        out_specs=[pl.BlockSpec((RB, H), lambda i: (i, 0)),
                   pl.BlockSpec((1, H), lambda i: (0, 0))],
        out_shape=[jax.ShapeDtypeStruct((NPAD, H), jnp.float32),
                   jax.ShapeDtypeStruct((1, H), jnp.float32)],
    )(p, b, s)


def _var_body(h_ref, sum_ref, var_ref):
    i = pl.program_id(0)
    mu = sum_ref[...] * (1.0 / N)
    d = h_ref[...] - mu
    npad_rows = NPAD - N
    ps = jnp.sum(d * d, 0, keepdims=True)

    @pl.when(i == 0)
    def _():
        var_ref[...] = ps

    @pl.when(i > 0)
    def _():
        var_ref[...] = var_ref[...] + ps

    @pl.when(i == 3)
    def _():
        var_ref[...] = (var_ref[...] - npad_rows * mu * mu) * (1.0 / N)


def _var(h, s):
    return pl.pallas_call(
        _var_body,
        grid=(4,),
        in_specs=[pl.BlockSpec((RB, H), lambda i: (i, 0)),
                  pl.BlockSpec((1, H), lambda i: (0, 0))],
        out_specs=pl.BlockSpec((1, H), lambda i: (0, 0)),
        out_shape=jax.ShapeDtypeStruct((1, H), jnp.float32),
    )(h, s)


def _bn_from_sums(sum_ref, var_ref, g_ref):
    mu = sum_ref[...] * (1.0 / N)
    scale = g_ref[...] * lax.rsqrt(var_ref[...] + 1e-5)
    return mu, scale


def _bnmm_body(h_ref, sum_ref, var_ref, g_ref, be_ref, w_ref, p_ref, q_ref):
    mu, scale = _bn_from_sums(sum_ref, var_ref, g_ref)
    hn = jnp.maximum((h_ref[...] - mu) * scale + be_ref[...], 0.0)
    pq = jnp.dot(hn, w_ref[...], preferred_element_type=jnp.float32, precision=lax.Precision.HIGHEST)
    p_ref[...] = pq[:, :H]
    q_ref[...] = pq[:, H:]


def _bnmm(h, s, v, g, be, wc):
    return pl.pallas_call(
        _bnmm_body,
        grid=(4,),
        in_specs=[pl.BlockSpec((RB, H), lambda i: (i, 0)),
                  pl.BlockSpec((1, H), lambda i: (0, 0)),
                  pl.BlockSpec((1, H), lambda i: (0, 0)),
                  pl.BlockSpec((1, H), lambda i: (0, 0)),
                  pl.BlockSpec((1, H), lambda i: (0, 0)),
                  pl.BlockSpec((H, 2 * H), lambda i: (0, 0))],
        out_specs=[pl.BlockSpec((RB, H), lambda i: (i, 0)),
                   pl.BlockSpec((RB, H), lambda i: (i, 0))],
        out_shape=[jax.ShapeDtypeStruct((NPAD, H), jnp.float32)] * 2,
    )(h, s, v, g, be, wc)


def _final_body(h_ref, sum_ref, var_ref, g_ref, be_ref, batch_ref, wl_ref,
                bl_ref, out_ref):
    mu, scale = _bn_from_sums(sum_ref, var_ref, g_ref)
    hn = (h_ref[...] - mu) * scale + be_ref[...]
    gids = lax.broadcasted_iota(jnp.int32, (G, NPAD), 0)
    m = (batch_ref[...] == gids).astype(jnp.float32)
    sums_g = jnp.dot(m, hn, preferred_element_type=jnp.float32, precision=lax.Precision.HIGHEST)
    counts = jnp.sum(m, axis=1, keepdims=True)
    pooled = sums_g / jnp.maximum(counts, 1.0)
    out = jnp.dot(pooled, wl_ref[...], preferred_element_type=jnp.float32, precision=lax.Precision.HIGHEST)
    out_ref[...] = jnp.maximum(out + bl_ref[...], 0.0)


def _final(h, s, v, g, be, batch2d, wlp, blv):
    return pl.pallas_call(
        _final_body,
        grid=(1,),
        in_specs=[pl.BlockSpec((NPAD, H), lambda i: (0, 0)),
                  pl.BlockSpec((1, H), lambda i: (0, 0)),
                  pl.BlockSpec((1, H), lambda i: (0, 0)),
                  pl.BlockSpec((1, H), lambda i: (0, 0)),
                  pl.BlockSpec((1, H), lambda i: (0, 0)),
                  pl.BlockSpec((1, NPAD), lambda i: (0, 0)),
                  pl.BlockSpec((H, H), lambda i: (0, 0)),
                  pl.BlockSpec((1, H), lambda i: (0, 0))],
        out_specs=pl.BlockSpec((G, H), lambda i: (0, 0)),
        out_shape=jax.ShapeDtypeStruct((G, H), jnp.float32),
    )(h, s, v, g, be, batch2d, wlp, blv)


def kernel(x, edge_index, batch, W1, b1, W2, b2, W3, b3,
           g1, be1, g2, be2, g3, be3, Wl, bl):
    x = x.astype(jnp.float32).reshape(-1, F_IN)
    src = edge_index[0].astype(jnp.int32)
    dst = edge_index[1].astype(jnp.int32)
    batch2d = jnp.pad(batch.astype(jnp.int32), (0, NPAD - N),
                      constant_values=G).reshape(1, NPAD)
    xp = jnp.pad(x, ((0, NPAD - N), (0, 0)))

    def split(w):
        f = w.shape[0] // 2
        return jnp.concatenate([w[:f] - w[f:], w[f:]], axis=1)

    wc1, wc2, wc3 = split(W1), split(W2), split(W3)
    b1r, b2r, b3r = (v.reshape(1, H) for v in (b1, b2, b3))
    g1r, g2r, g3r = (v.reshape(1, H) for v in (g1, g2, g3))
    be1r, be2r, be3r = (v.reshape(1, H) for v in (be1, be2, be3))
    wlp = jnp.zeros((H, H), jnp.float32).at[:, 0].set(Wl[:, 0])
    blv = jnp.zeros((1, H), jnp.float32).at[0, 0].set(bl[0])

    srcl, dstl, cnts = _sc_filter(src, dst)

    p1, q1 = _mm_in(xp, wc1)
    s1 = _sc_segmax(q1, srcl, dstl, cnts)
    h1, sum1 = _stats(p1, b1r, s1)
    var1 = _var(h1, sum1)

    p2, q2 = _bnmm(h1, sum1, var1, g1r, be1r, wc2)
    s2 = _sc_segmax(q2, srcl, dstl, cnts)
    h2, sum2 = _stats(p2, b2r, s2)
    var2 = _var(h2, sum2)

    p3, q3 = _bnmm(h2, sum2, var2, g2r, be2r, wc3)
    s3 = _sc_segmax(q3, srcl, dstl, cnts)
    h3, sum3 = _stats(p3, b3r, s3)
    var3 = _var(h3, sum3)

    out = _final(h3, sum3, var3, g3r, be3r, batch2d, wlp, blv)
    return out[:, :1]


# speedup vs baseline: 1.2622x; 1.2622x over previous
import functools

import jax
import jax.numpy as jnp
from jax import lax
from jax.experimental import pallas as pl
from jax.experimental.pallas import tpu as pltpu
from jax.experimental.pallas import tpu_sc as plsc

N = 10000
E = 320000
F_IN = 24
H = 128
G = 64

NT = 32
NPT = 320
NPAD = NT * NPT
RB = NPAD // 4

CH = 128
PADM = 2 * CH
FLUSH = 2048
STG = FLUSH + PADM + 64
TRASH = STG - 16
ECAP = E + 2 * FLUSH
NEG = -1.0e30

_mesh = plsc.VectorSubcoreMesh(core_axis_name="c", subcore_axis_name="s")
_sc_params = pltpu.CompilerParams(needs_layout_passes=False)


def _wid():
    return lax.axis_index("s") * 2 + lax.axis_index("c")


@functools.partial(
    pl.kernel,
    mesh=_mesh,
    compiler_params=_sc_params,
    out_type=[
        jax.ShapeDtypeStruct((NT * ECAP,), jnp.int32),
        jax.ShapeDtypeStruct((NT * ECAP,), jnp.int32),
        jax.ShapeDtypeStruct((NT * 16,), jnp.int32),
    ],
    scratch_types=[
        pltpu.VMEM((4000,), jnp.int32),
        pltpu.VMEM((4000,), jnp.int32),
        pltpu.VMEM((STG,), jnp.int32),
        pltpu.VMEM((STG,), jnp.int32),
        pltpu.VMEM((16,), jnp.int32),
    ],
)
def _sc_filter(src_hbm, dst_hbm, srcl_hbm, dstl_hbm, cnt_hbm,
               dchunk, schunk, stg_s, stg_d, cvec):
    wid = _wid()
    base = wid * NPT
    lbase = pl.multiple_of(wid * ECAP, 8)
    lanes = lax.iota(jnp.int32, 16)

    def inner(j, carry):
        off, hoff = carry
        dv1 = dchunk[pl.ds(j * 32, 16)]
        sv1 = schunk[pl.ds(j * 32, 16)]
        dv2 = dchunk[pl.ds(j * 32 + 16, 16)]
        sv2 = schunk[pl.ds(j * 32 + 16, 16)]
        dl1 = dv1 - base
        dl2 = dv2 - base
        m1 = (dl1 >= 0) & (dl1 < NPT)
        m2 = (dl2 >= 0) & (dl2 < NPT)
        mi1 = jnp.where(m1, 1, 0)
        mi2 = jnp.where(m2, 1, 0)
        i1 = plsc.cumsum(mi1)
        i2 = plsc.cumsum(mi2)
        c1 = i1[15]
        pos1 = jnp.where(m1, off + (i1 - mi1), TRASH + lanes)
        pos2 = jnp.where(m2, off + c1 + (i2 - mi2), TRASH + lanes)
        plsc.store_scatter(stg_s, [pos1], sv1)
        plsc.store_scatter(stg_d, [pos1], dl1)
        plsc.store_scatter(stg_s, [pos2], sv2)
        plsc.store_scatter(stg_d, [pos2], dl2)
        off = off + c1 + i2[15]

        def do_flush(o, h):
            hb = pl.multiple_of(lbase + h, 8)
            pltpu.sync_copy(stg_s.at[pl.ds(0, FLUSH)],
                            srcl_hbm.at[pl.ds(hb, FLUSH)])
            pltpu.sync_copy(stg_d.at[pl.ds(0, FLUSH)],
                            dstl_hbm.at[pl.ds(hb, FLUSH)])
            stg_s[pl.ds(0, 16)] = stg_s[pl.ds(FLUSH, 16)]
            stg_s[pl.ds(16, 16)] = stg_s[pl.ds(FLUSH + 16, 16)]
            stg_d[pl.ds(0, 16)] = stg_d[pl.ds(FLUSH, 16)]
            stg_d[pl.ds(16, 16)] = stg_d[pl.ds(FLUSH + 16, 16)]
            return o - FLUSH, h + FLUSH

        return lax.cond(off >= FLUSH, do_flush, lambda o, h: (o, h), off, hoff)

    def chunk_body(ci, carry):
        pltpu.sync_copy(src_hbm.at[pl.ds(ci * 4000, 4000)], schunk)
        pltpu.sync_copy(dst_hbm.at[pl.ds(ci * 4000, 4000)], dchunk)
        return lax.fori_loop(0, 125, inner, carry)

    off, hoff = lax.fori_loop(0, E // 4000, chunk_body, (0, 0))

    a0 = (off // 16) * 16
    rem = off - a0
    keep_s = stg_s[pl.ds(a0, 16)]
    keep_d = stg_d[pl.ds(a0, 16)]
    stg_s[pl.ds(a0, 16)] = jnp.where(lanes < rem, keep_s, 0)
    stg_d[pl.ds(a0, 16)] = jnp.where(lanes < rem, keep_d, NPT)
    for k in range(1, PADM // 16):
        stg_s[pl.ds(a0 + 16 * k, 16)] = jnp.zeros((16,), jnp.int32)
        stg_d[pl.ds(a0 + 16 * k, 16)] = jnp.full((16,), NPT, jnp.int32)
    hb = pl.multiple_of(lbase + hoff, 8)
    pltpu.sync_copy(stg_s.at[pl.ds(0, FLUSH)], srcl_hbm.at[pl.ds(hb, FLUSH)])
    pltpu.sync_copy(stg_d.at[pl.ds(0, FLUSH)], dstl_hbm.at[pl.ds(hb, FLUSH)])

    total = hoff + ((off + PADM - 1) // PADM) * PADM
    cvec[...] = jnp.broadcast_to(total, (16,)).astype(jnp.int32)
    pltpu.sync_copy(cvec, cnt_hbm.at[pl.ds(pl.multiple_of(wid * 16, 8), 16)])


@functools.partial(
    pl.kernel,
    mesh=_mesh,
    compiler_params=_sc_params,
    out_type=jax.ShapeDtypeStruct((NPAD, H), jnp.float32),
    scratch_types=[
        pltpu.VMEM((NPT + 1, H), jnp.float32),
        pltpu.VMEM((2, CH), jnp.int32),
        pltpu.VMEM((2, CH, H), jnp.float32),
        pltpu.VMEM((2, CH), jnp.int32),
        pltpu.VMEM((16,), jnp.int32),
        pltpu.SemaphoreType.DMA,
        pltpu.SemaphoreType.DMA,
    ],
)
def _sc_segmax(q_hbm, srcl_hbm, dstl_hbm, cnt_hbm, s_hbm,
               s_loc, idx_v, rows_v, dl_vmem, cnt_vmem, sem0, sem1):
    wid = _wid()
    base = pl.multiple_of(wid * NPT, 8)
    lbase = pl.multiple_of(wid * ECAP, 8)
    pltpu.sync_copy(cnt_hbm.at[pl.ds(pl.multiple_of(wid * 16, 8), 16)], cnt_vmem)
    n = cnt_vmem[...][0]
    ng = n // CH

    negv = jnp.full((16,), NEG, jnp.float32)

    def initb(i, _):
        for c in range(H // 16):
            s_loc[i, pl.ds(c * 16, 16)] = negv
        return 0

    lax.fori_loop(0, NPT + 1, initb, 0)

    def start(slot, sem, g):
        gb = pl.multiple_of(lbase + g * CH, 8)
        pltpu.sync_copy(srcl_hbm.at[pl.ds(gb, CH)], idx_v.at[slot])
        pltpu.async_copy(q_hbm.at[idx_v.at[slot]], rows_v.at[slot], sem)
        pltpu.async_copy(dstl_hbm.at[pl.ds(gb, CH)], dl_vmem.at[slot], sem)

    def wait(slot, sem):
        pltpu.make_async_copy(q_hbm.at[pl.ds(0, CH)], rows_v.at[slot], sem).wait()
        pltpu.make_async_copy(dstl_hbm.at[pl.ds(0, CH)], dl_vmem.at[slot],
                              sem).wait()

    def drain(slot, g):
        def group_body(gi, _):
            dl16 = dl_vmem[slot, pl.ds(gi * 16, 16)]
            for t in range(16):
                d = dl16[t]
                e = gi * 16 + t
                for c in range(H // 16):
                    sl = pl.ds(c * 16, 16)
                    s_loc[d, sl] = jnp.maximum(s_loc[d, sl],
                                               rows_v[slot, e, sl])
            return 0

        lax.fori_loop(0, CH // 16, group_body, 0, unroll=2)

    @pl.when(ng > 0)
    def _():
        start(0, sem0, 0)

    def body(i, _):
        g0 = 2 * i
        start(1, sem1, g0 + 1)
        wait(0, sem0)
        drain(0, g0)

        @pl.when(g0 + 2 < ng)
        def _():
            start(0, sem0, g0 + 2)

        wait(1, sem1)
        drain(1, g0 + 1)
        return 0

    lax.fori_loop(0, ng // 2, body, 0)
    pltpu.sync_copy(s_loc.at[pl.ds(0, NPT)], s_hbm.at[pl.ds(base, NPT)])


def _mm_in_body(x_ref, w_ref, p_ref, q_ref):
    pq = jnp.dot(x_ref[...], w_ref[...], preferred_element_type=jnp.float32, precision=lax.Precision.HIGHEST)
    p_ref[...] = pq[:, :H]
    q_ref[...] = pq[:, H:]


def _mm_in(xp, wc):
    return pl.pallas_call(
        _mm_in_body,
        grid=(4,),
        in_specs=[pl.BlockSpec((RB, F_IN), lambda i: (i, 0)),
                  pl.BlockSpec((F_IN, 2 * H), lambda i: (0, 0))],
        out_specs=[pl.BlockSpec((RB, H), lambda i: (i, 0)),
                   pl.BlockSpec((RB, H), lambda i: (i, 0))],
        out_shape=[jax.ShapeDtypeStruct((NPAD, H), jnp.float32)] * 2,
    )(xp, wc)


def _stats_body(p_ref, b_ref, s_ref, h_ref, sums_ref):
    i = pl.program_id(0)
    h = jnp.maximum(p_ref[...] + b_ref[...] + s_ref[...], 0.0)
    h_ref[...] = h
    ps = jnp.sum(h, 0, keepdims=True)

    @pl.when(i == 0)
    def _():
        sums_ref[...] = ps

    @pl.when(i > 0)
    def _():
        sums_ref[...] = sums_ref[...] + ps


def _stats(p, b, s):
    return pl.pallas_call(
        _stats_body,
        grid=(4,),
        in_specs=[pl.BlockSpec((RB, H), lambda i: (i, 0)),
                  pl.BlockSpec((1, H), lambda i: (0, 0)),
                  pl.BlockSpec((RB, H), lambda i: (i, 0))],
        out_specs=[pl.BlockSpec((RB, H), lambda i: (i, 0)),
                   pl.BlockSpec((1, H), lambda i: (0, 0))],
        out_shape=[jax.ShapeDtypeStruct((NPAD, H), jnp.float32),
                   jax.ShapeDtypeStruct((1, H), jnp.float32)],
    )(p, b, s)


def _var_body(h_ref, sum_ref, var_ref):
    i = pl.program_id(0)
    mu = sum_ref[...] * (1.0 / N)
    d = h_ref[...] - mu
    npad_rows = NPAD - N
    ps = jnp.sum(d * d, 0, keepdims=True)

    @pl.when(i == 0)
    def _():
        var_ref[...] = ps

    @pl.when(i > 0)
    def _():
        var_ref[...] = var_ref[...] + ps

    @pl.when(i == 3)
    def _():
        var_ref[...] = (var_ref[...] - npad_rows * mu * mu) * (1.0 / N)


def _var(h, s):
    return pl.pallas_call(
        _var_body,
        grid=(4,),
        in_specs=[pl.BlockSpec((RB, H), lambda i: (i, 0)),
                  pl.BlockSpec((1, H), lambda i: (0, 0))],
        out_specs=pl.BlockSpec((1, H), lambda i: (0, 0)),
        out_shape=jax.ShapeDtypeStruct((1, H), jnp.float32),
    )(h, s)


def _bn_from_sums(sum_ref, var_ref, g_ref):
    mu = sum_ref[...] * (1.0 / N)
    scale = g_ref[...] * lax.rsqrt(var_ref[...] + 1e-5)
    return mu, scale


def _bnmm_body(h_ref, sum_ref, var_ref, g_ref, be_ref, w_ref, p_ref, q_ref):
    mu, scale = _bn_from_sums(sum_ref, var_ref, g_ref)
    hn = jnp.maximum((h_ref[...] - mu) * scale + be_ref[...], 0.0)
    pq = jnp.dot(hn, w_ref[...], preferred_element_type=jnp.float32, precision=lax.Precision.HIGHEST)
    p_ref[...] = pq[:, :H]
    q_ref[...] = pq[:, H:]


def _bnmm(h, s, v, g, be, wc):
    return pl.pallas_call(
        _bnmm_body,
        grid=(4,),
        in_specs=[pl.BlockSpec((RB, H), lambda i: (i, 0)),
                  pl.BlockSpec((1, H), lambda i: (0, 0)),
                  pl.BlockSpec((1, H), lambda i: (0, 0)),
                  pl.BlockSpec((1, H), lambda i: (0, 0)),
                  pl.BlockSpec((1, H), lambda i: (0, 0)),
                  pl.BlockSpec((H, 2 * H), lambda i: (0, 0))],
        out_specs=[pl.BlockSpec((RB, H), lambda i: (i, 0)),
                   pl.BlockSpec((RB, H), lambda i: (i, 0))],
        out_shape=[jax.ShapeDtypeStruct((NPAD, H), jnp.float32)] * 2,
    )(h, s, v, g, be, wc)


def _final_body(h_ref, sum_ref, var_ref, g_ref, be_ref, batch_ref, wl_ref,
                bl_ref, out_ref):
    mu, scale = _bn_from_sums(sum_ref, var_ref, g_ref)
    hn = (h_ref[...] - mu) * scale + be_ref[...]
    gids = lax.broadcasted_iota(jnp.int32, (G, NPAD), 0)
    m = (batch_ref[...] == gids).astype(jnp.float32)
    sums_g = jnp.dot(m, hn, preferred_element_type=jnp.float32, precision=lax.Precision.HIGHEST)
    counts = jnp.sum(m, axis=1, keepdims=True)
    pooled = sums_g / jnp.maximum(counts, 1.0)
    out = jnp.dot(pooled, wl_ref[...], preferred_element_type=jnp.float32, precision=lax.Precision.HIGHEST)
    out_ref[...] = jnp.maximum(out + bl_ref[...], 0.0)


def _final(h, s, v, g, be, batch2d, wlp, blv):
    return pl.pallas_call(
        _final_body,
        grid=(1,),
        in_specs=[pl.BlockSpec((NPAD, H), lambda i: (0, 0)),
                  pl.BlockSpec((1, H), lambda i: (0, 0)),
                  pl.BlockSpec((1, H), lambda i: (0, 0)),
                  pl.BlockSpec((1, H), lambda i: (0, 0)),
                  pl.BlockSpec((1, H), lambda i: (0, 0)),
                  pl.BlockSpec((1, NPAD), lambda i: (0, 0)),
                  pl.BlockSpec((H, H), lambda i: (0, 0)),
                  pl.BlockSpec((1, H), lambda i: (0, 0))],
        out_specs=pl.BlockSpec((G, H), lambda i: (0, 0)),
        out_shape=jax.ShapeDtypeStruct((G, H), jnp.float32),
    )(h, s, v, g, be, batch2d, wlp, blv)


def kernel(x, edge_index, batch, W1, b1, W2, b2, W3, b3,
           g1, be1, g2, be2, g3, be3, Wl, bl):
    x = x.astype(jnp.float32).reshape(-1, F_IN)
    src = edge_index[0].astype(jnp.int32)
    dst = edge_index[1].astype(jnp.int32)
    batch2d = jnp.pad(batch.astype(jnp.int32), (0, NPAD - N),
                      constant_values=G).reshape(1, NPAD)
    xp = jnp.pad(x, ((0, NPAD - N), (0, 0)))

    def split(w):
        f = w.shape[0] // 2
        return jnp.concatenate([w[:f] - w[f:], w[f:]], axis=1)

    wc1, wc2, wc3 = split(W1), split(W2), split(W3)
    b1r, b2r, b3r = (v.reshape(1, H) for v in (b1, b2, b3))
    g1r, g2r, g3r = (v.reshape(1, H) for v in (g1, g2, g3))
    be1r, be2r, be3r = (v.reshape(1, H) for v in (be1, be2, be3))
    wlp = jnp.zeros((H, H), jnp.float32).at[:, 0].set(Wl[:, 0])
    blv = jnp.zeros((1, H), jnp.float32).at[0, 0].set(bl[0])

    srcl, dstl, cnts = _sc_filter(src, dst)

    p1, q1 = _mm_in(xp, wc1)
    s1 = _sc_segmax(q1, srcl, dstl, cnts)
    h1, sum1 = _stats(p1, b1r, s1)
    var1 = _var(h1, sum1)

    p2, q2 = _bnmm(h1, sum1, var1, g1r, be1r, wc2)
    s2 = _sc_segmax(q2, srcl, dstl, cnts)
    h2, sum2 = _stats(p2, b2r, s2)
    var2 = _var(h2, sum2)

    p3, q3 = _bnmm(h2, sum2, var2, g2r, be2r, wc3)
    s3 = _sc_segmax(q3, srcl, dstl, cnts)
    h3, sum3 = _stats(p3, b3r, s3)
    var3 = _var(h3, sum3)

    out = _final(h3, sum3, var3, g3r, be3r, batch2d, wlp, blv)
    return out[:, :1]


# speedup vs baseline: 1.3784x; 1.0921x over previous
import functools

import jax
import jax.numpy as jnp
from jax import lax
from jax.experimental import pallas as pl
from jax.experimental.pallas import tpu as pltpu
from jax.experimental.pallas import tpu_sc as plsc

N = 10000
E = 320000
F_IN = 24
H = 128
G = 64

NT = 32
NPT = 320
NPAD = NT * NPT
RB = NPAD // 4

CH = 128
PADM = 2 * CH
FLUSH = 2048
FCH = 8000
STG = FLUSH + PADM + 128
TRASH = STG - 16
ECAP = E + 2 * FLUSH
NEG = -1.0e30

_mesh = plsc.VectorSubcoreMesh(core_axis_name="c", subcore_axis_name="s")
_sc_params = pltpu.CompilerParams(needs_layout_passes=False)


def _wid():
    return lax.axis_index("s") * 2 + lax.axis_index("c")


@functools.partial(
    pl.kernel,
    mesh=_mesh,
    compiler_params=_sc_params,
    out_type=[
        jax.ShapeDtypeStruct((NT * ECAP,), jnp.int32),
        jax.ShapeDtypeStruct((NT * ECAP,), jnp.int32),
        jax.ShapeDtypeStruct((NT * 16,), jnp.int32),
    ],
    scratch_types=[
        pltpu.VMEM((FCH,), jnp.int32),
        pltpu.VMEM((FCH,), jnp.int32),
        pltpu.VMEM((STG,), jnp.int32),
        pltpu.VMEM((STG,), jnp.int32),
        pltpu.VMEM((16,), jnp.int32),
    ],
)
def _sc_filter(src_hbm, dst_hbm, srcl_hbm, dstl_hbm, cnt_hbm,
               dchunk, schunk, stg_s, stg_d, cvec):
    wid = _wid()
    base = wid * NPT
    lbase = pl.multiple_of(wid * ECAP, 8)
    lanes = lax.iota(jnp.int32, 16)

    def inner(j, carry):
        off, hoff = carry
        dvs, svs, mis, incls = [], [], [], []
        for u in range(4):
            dv = dchunk[pl.ds(j * 64 + 16 * u, 16)]
            sv = schunk[pl.ds(j * 64 + 16 * u, 16)]
            dl = dv - base
            m = (dl >= 0) & (dl < NPT)
            mi = jnp.where(m, 1, 0)
            dvs.append(dl)
            svs.append(sv)
            mis.append(mi)
            incls.append(plsc.cumsum(mi))
        o = off
        for u in range(4):
            m = mis[u] > 0
            pos = jnp.where(m, o + (incls[u] - mis[u]), TRASH + lanes)
            plsc.store_scatter(stg_s, [pos], svs[u])
            plsc.store_scatter(stg_d, [pos], dvs[u])
            o = o + incls[u][15]
        off = o

        def do_flush(o, h):
            hb = pl.multiple_of(lbase + h, 8)
            pltpu.sync_copy(stg_s.at[pl.ds(0, FLUSH)],
                            srcl_hbm.at[pl.ds(hb, FLUSH)])
            pltpu.sync_copy(stg_d.at[pl.ds(0, FLUSH)],
                            dstl_hbm.at[pl.ds(hb, FLUSH)])
            for u in range(4):
                stg_s[pl.ds(16 * u, 16)] = stg_s[pl.ds(FLUSH + 16 * u, 16)]
                stg_d[pl.ds(16 * u, 16)] = stg_d[pl.ds(FLUSH + 16 * u, 16)]
            return o - FLUSH, h + FLUSH

        return lax.cond(off >= FLUSH, do_flush, lambda o, h: (o, h), off, hoff)

    def chunk_body(ci, carry):
        pltpu.sync_copy(src_hbm.at[pl.ds(ci * FCH, FCH)], schunk)
        pltpu.sync_copy(dst_hbm.at[pl.ds(ci * FCH, FCH)], dchunk)
        return lax.fori_loop(0, FCH // 64, inner, carry)

    off, hoff = lax.fori_loop(0, E // FCH, chunk_body, (0, 0))

    a0 = (off // 16) * 16
    rem = off - a0
    keep_s = stg_s[pl.ds(a0, 16)]
    keep_d = stg_d[pl.ds(a0, 16)]
    stg_s[pl.ds(a0, 16)] = jnp.where(lanes < rem, keep_s, 0)
    stg_d[pl.ds(a0, 16)] = jnp.where(lanes < rem, keep_d, NPT)
    for k in range(1, PADM // 16):
        stg_s[pl.ds(a0 + 16 * k, 16)] = jnp.zeros((16,), jnp.int32)
        stg_d[pl.ds(a0 + 16 * k, 16)] = jnp.full((16,), NPT, jnp.int32)
    hb = pl.multiple_of(lbase + hoff, 8)
    pltpu.sync_copy(stg_s.at[pl.ds(0, FLUSH)], srcl_hbm.at[pl.ds(hb, FLUSH)])
    pltpu.sync_copy(stg_d.at[pl.ds(0, FLUSH)], dstl_hbm.at[pl.ds(hb, FLUSH)])

    total = hoff + ((off + PADM - 1) // PADM) * PADM
    cvec[...] = jnp.broadcast_to(total, (16,)).astype(jnp.int32)
    pltpu.sync_copy(cvec, cnt_hbm.at[pl.ds(pl.multiple_of(wid * 16, 8), 16)])


@functools.partial(
    pl.kernel,
    mesh=_mesh,
    compiler_params=_sc_params,
    out_type=jax.ShapeDtypeStruct((NPAD, H), jnp.float32),
    scratch_types=[
        pltpu.VMEM((NPT + 1, H), jnp.float32),
        pltpu.VMEM((2, CH), jnp.int32),
        pltpu.VMEM((2, CH, H), jnp.float32),
        pltpu.VMEM((2, CH), jnp.int32),
        pltpu.VMEM((16,), jnp.int32),
        pltpu.SemaphoreType.DMA,
        pltpu.SemaphoreType.DMA,
    ],
)
def _sc_segmax(q_hbm, srcl_hbm, dstl_hbm, cnt_hbm, s_hbm,
               s_loc, idx_v, rows_v, dl_vmem, cnt_vmem, sem0, sem1):
    wid = _wid()
    base = pl.multiple_of(wid * NPT, 8)
    lbase = pl.multiple_of(wid * ECAP, 8)
    pltpu.sync_copy(cnt_hbm.at[pl.ds(pl.multiple_of(wid * 16, 8), 16)], cnt_vmem)
    n = cnt_vmem[...][0]
    ng = n // CH

    negv = jnp.full((16,), NEG, jnp.float32)

    def initb(i, _):
        for c in range(H // 16):
            s_loc[i, pl.ds(c * 16, 16)] = negv
        return 0

    lax.fori_loop(0, NPT + 1, initb, 0)

    def start(slot, sem, g):
        gb = pl.multiple_of(lbase + g * CH, 8)
        pltpu.sync_copy(srcl_hbm.at[pl.ds(gb, CH)], idx_v.at[slot])
        pltpu.async_copy(q_hbm.at[idx_v.at[slot]], rows_v.at[slot], sem)
        pltpu.async_copy(dstl_hbm.at[pl.ds(gb, CH)], dl_vmem.at[slot], sem)

    def wait(slot, sem):
        pltpu.make_async_copy(q_hbm.at[pl.ds(0, CH)], rows_v.at[slot], sem).wait()
        pltpu.make_async_copy(dstl_hbm.at[pl.ds(0, CH)], dl_vmem.at[slot],
                              sem).wait()

    def drain(slot, g):
        def group_body(gi, _):
            dl16 = dl_vmem[slot, pl.ds(gi * 16, 16)]
            for t in range(16):
                d = dl16[t]
                e = gi * 16 + t
                for c in range(H // 16):
                    sl = pl.ds(c * 16, 16)
                    s_loc[d, sl] = jnp.maximum(s_loc[d, sl],
                                               rows_v[slot, e, sl])
            return 0

        lax.fori_loop(0, CH // 16, group_body, 0, unroll=2)

    @pl.when(ng > 0)
    def _():
        start(0, sem0, 0)

    def body(i, _):
        g0 = 2 * i
        start(1, sem1, g0 + 1)
        wait(0, sem0)
        drain(0, g0)

        @pl.when(g0 + 2 < ng)
        def _():
            start(0, sem0, g0 + 2)

        wait(1, sem1)
        drain(1, g0 + 1)
        return 0

    lax.fori_loop(0, ng // 2, body, 0)
    pltpu.sync_copy(s_loc.at[pl.ds(0, NPT)], s_hbm.at[pl.ds(base, NPT)])


def _mm_in_body(x_ref, w_ref, p_ref, q_ref):
    pq = jnp.dot(x_ref[...], w_ref[...], preferred_element_type=jnp.float32, precision=lax.Precision.HIGHEST)
    p_ref[...] = pq[:, :H]
    q_ref[...] = pq[:, H:]


def _mm_in(xp, wc):
    return pl.pallas_call(
        _mm_in_body,
        grid=(4,),
        in_specs=[pl.BlockSpec((RB, F_IN), lambda i: (i, 0)),
                  pl.BlockSpec((F_IN, 2 * H), lambda i: (0, 0))],
        out_specs=[pl.BlockSpec((RB, H), lambda i: (i, 0)),
                   pl.BlockSpec((RB, H), lambda i: (i, 0))],
        out_shape=[jax.ShapeDtypeStruct((NPAD, H), jnp.float32)] * 2,
    )(xp, wc)


def _stats_body(p_ref, b_ref, s_ref, h_ref, sums_ref):
    i = pl.program_id(0)
    h = jnp.maximum(p_ref[...] + b_ref[...] + s_ref[...], 0.0)
    h_ref[...] = h
    ps = jnp.sum(h, 0, keepdims=True)

    @pl.when(i == 0)
    def _():
        sums_ref[...] = ps

    @pl.when(i > 0)
    def _():
        sums_ref[...] = sums_ref[...] + ps


def _stats(p, b, s):
    return pl.pallas_call(
        _stats_body,
        grid=(4,),
        in_specs=[pl.BlockSpec((RB, H), lambda i: (i, 0)),
                  pl.BlockSpec((1, H), lambda i: (0, 0)),
                  pl.BlockSpec((RB, H), lambda i: (i, 0))],
        out_specs=[pl.BlockSpec((RB, H), lambda i: (i, 0)),
                   pl.BlockSpec((1, H), lambda i: (0, 0))],
        out_shape=[jax.ShapeDtypeStruct((NPAD, H), jnp.float32),
                   jax.ShapeDtypeStruct((1, H), jnp.float32)],
    )(p, b, s)


def _var_body(h_ref, sum_ref, var_ref):
    i = pl.program_id(0)
    mu = sum_ref[...] * (1.0 / N)
    d = h_ref[...] - mu
    npad_rows = NPAD - N
    ps = jnp.sum(d * d, 0, keepdims=True)

    @pl.when(i == 0)
    def _():
        var_ref[...] = ps

    @pl.when(i > 0)
    def _():
        var_ref[...] = var_ref[...] + ps

    @pl.when(i == 3)
    def _():
        var_ref[...] = (var_ref[...] - npad_rows * mu * mu) * (1.0 / N)


def _var(h, s):
    return pl.pallas_call(
        _var_body,
        grid=(4,),
        in_specs=[pl.BlockSpec((RB, H), lambda i: (i, 0)),
                  pl.BlockSpec((1, H), lambda i: (0, 0))],
        out_specs=pl.BlockSpec((1, H), lambda i: (0, 0)),
        out_shape=jax.ShapeDtypeStruct((1, H), jnp.float32),
    )(h, s)


def _bn_from_sums(sum_ref, var_ref, g_ref):
    mu = sum_ref[...] * (1.0 / N)
    scale = g_ref[...] * lax.rsqrt(var_ref[...] + 1e-5)
    return mu, scale


def _bnmm_body(h_ref, sum_ref, var_ref, g_ref, be_ref, w_ref, p_ref, q_ref):
    mu, scale = _bn_from_sums(sum_ref, var_ref, g_ref)
    hn = jnp.maximum((h_ref[...] - mu) * scale + be_ref[...], 0.0)
    pq = jnp.dot(hn, w_ref[...], preferred_element_type=jnp.float32, precision=lax.Precision.HIGHEST)
    p_ref[...] = pq[:, :H]
    q_ref[...] = pq[:, H:]


def _bnmm(h, s, v, g, be, wc):
    return pl.pallas_call(
        _bnmm_body,
        grid=(4,),
        in_specs=[pl.BlockSpec((RB, H), lambda i: (i, 0)),
                  pl.BlockSpec((1, H), lambda i: (0, 0)),
                  pl.BlockSpec((1, H), lambda i: (0, 0)),
                  pl.BlockSpec((1, H), lambda i: (0, 0)),
                  pl.BlockSpec((1, H), lambda i: (0, 0)),
                  pl.BlockSpec((H, 2 * H), lambda i: (0, 0))],
        out_specs=[pl.BlockSpec((RB, H), lambda i: (i, 0)),
                   pl.BlockSpec((RB, H), lambda i: (i, 0))],
        out_shape=[jax.ShapeDtypeStruct((NPAD, H), jnp.float32)] * 2,
    )(h, s, v, g, be, wc)


def _final_body(h_ref, sum_ref, var_ref, g_ref, be_ref, batch_ref, wl_ref,
                bl_ref, out_ref):
    mu, scale = _bn_from_sums(sum_ref, var_ref, g_ref)
    hn = (h_ref[...] - mu) * scale + be_ref[...]
    gids = lax.broadcasted_iota(jnp.int32, (G, NPAD), 0)
    m = (batch_ref[...] == gids).astype(jnp.float32)
    sums_g = jnp.dot(m, hn, preferred_element_type=jnp.float32, precision=lax.Precision.HIGHEST)
    counts = jnp.sum(m, axis=1, keepdims=True)
    pooled = sums_g / jnp.maximum(counts, 1.0)
    out = jnp.dot(pooled, wl_ref[...], preferred_element_type=jnp.float32, precision=lax.Precision.HIGHEST)
    out_ref[...] = jnp.maximum(out + bl_ref[...], 0.0)


def _final(h, s, v, g, be, batch2d, wlp, blv):
    return pl.pallas_call(
        _final_body,
        grid=(1,),
        in_specs=[pl.BlockSpec((NPAD, H), lambda i: (0, 0)),
                  pl.BlockSpec((1, H), lambda i: (0, 0)),
                  pl.BlockSpec((1, H), lambda i: (0, 0)),
                  pl.BlockSpec((1, H), lambda i: (0, 0)),
                  pl.BlockSpec((1, H), lambda i: (0, 0)),
                  pl.BlockSpec((1, NPAD), lambda i: (0, 0)),
                  pl.BlockSpec((H, H), lambda i: (0, 0)),
                  pl.BlockSpec((1, H), lambda i: (0, 0))],
        out_specs=pl.BlockSpec((G, H), lambda i: (0, 0)),
        out_shape=jax.ShapeDtypeStruct((G, H), jnp.float32),
    )(h, s, v, g, be, batch2d, wlp, blv)


def kernel(x, edge_index, batch, W1, b1, W2, b2, W3, b3,
           g1, be1, g2, be2, g3, be3, Wl, bl):
    x = x.astype(jnp.float32).reshape(-1, F_IN)
    src = edge_index[0].astype(jnp.int32)
    dst = edge_index[1].astype(jnp.int32)
    batch2d = jnp.pad(batch.astype(jnp.int32), (0, NPAD - N),
                      constant_values=G).reshape(1, NPAD)
    xp = jnp.pad(x, ((0, NPAD - N), (0, 0)))

    def split(w):
        f = w.shape[0] // 2
        return jnp.concatenate([w[:f] - w[f:], w[f:]], axis=1)

    wc1, wc2, wc3 = split(W1), split(W2), split(W3)
    b1r, b2r, b3r = (v.reshape(1, H) for v in (b1, b2, b3))
    g1r, g2r, g3r = (v.reshape(1, H) for v in (g1, g2, g3))
    be1r, be2r, be3r = (v.reshape(1, H) for v in (be1, be2, be3))
    wlp = jnp.zeros((H, H), jnp.float32).at[:, 0].set(Wl[:, 0])
    blv = jnp.zeros((1, H), jnp.float32).at[0, 0].set(bl[0])

    srcl, dstl, cnts = _sc_filter(src, dst)

    p1, q1 = _mm_in(xp, wc1)
    s1 = _sc_segmax(q1, srcl, dstl, cnts)
    h1, sum1 = _stats(p1, b1r, s1)
    var1 = _var(h1, sum1)

    p2, q2 = _bnmm(h1, sum1, var1, g1r, be1r, wc2)
    s2 = _sc_segmax(q2, srcl, dstl, cnts)
    h2, sum2 = _stats(p2, b2r, s2)
    var2 = _var(h2, sum2)

    p3, q3 = _bnmm(h2, sum2, var2, g2r, be2r, wc3)
    s3 = _sc_segmax(q3, srcl, dstl, cnts)
    h3, sum3 = _stats(p3, b3r, s3)
    var3 = _var(h3, sum3)

    out = _final(h3, sum3, var3, g3r, be3r, batch2d, wlp, blv)
    return out[:, :1]


# speedup vs baseline: 1.4218x; 1.0315x over previous
import functools

import jax
import jax.numpy as jnp
from jax import lax
from jax.experimental import pallas as pl
from jax.experimental.pallas import tpu as pltpu
from jax.experimental.pallas import tpu_sc as plsc

N = 10000
E = 320000
F_IN = 24
H = 128
G = 64

NT = 32
NPT = 320
NPAD = NT * NPT
RB = NPAD // 4

CH = 128
PADM = 2 * CH
FLUSH = 2048
FCH = 8000
STG = FLUSH + PADM + 128
TRASH = STG - 16
ECAP = E + 2 * FLUSH
NEG = -1.0e30

_mesh = plsc.VectorSubcoreMesh(core_axis_name="c", subcore_axis_name="s")
_sc_params = pltpu.CompilerParams(needs_layout_passes=False)


def _wid():
    return lax.axis_index("s") * 2 + lax.axis_index("c")


@functools.partial(
    pl.kernel,
    mesh=_mesh,
    compiler_params=_sc_params,
    out_type=[
        jax.ShapeDtypeStruct((NT * ECAP,), jnp.int32),
        jax.ShapeDtypeStruct((NT * ECAP,), jnp.int32),
        jax.ShapeDtypeStruct((NT * 16,), jnp.int32),
    ],
    scratch_types=[
        pltpu.VMEM((FCH,), jnp.int32),
        pltpu.VMEM((FCH,), jnp.int32),
        pltpu.VMEM((STG,), jnp.int32),
        pltpu.VMEM((STG,), jnp.int32),
        pltpu.VMEM((16,), jnp.int32),
    ],
)
def _sc_filter(src_hbm, dst_hbm, srcl_hbm, dstl_hbm, cnt_hbm,
               dchunk, schunk, stg_s, stg_d, cvec):
    wid = _wid()
    base = wid * NPT
    lbase = pl.multiple_of(wid * ECAP, 8)
    lanes = lax.iota(jnp.int32, 16)

    def inner(j, carry):
        off, hoff = carry
        dvs, svs, mis, incls = [], [], [], []
        for u in range(4):
            dv = dchunk[pl.ds(j * 64 + 16 * u, 16)]
            sv = schunk[pl.ds(j * 64 + 16 * u, 16)]
            dl = dv - base
            m = (dl >= 0) & (dl < NPT)
            mi = jnp.where(m, 1, 0)
            dvs.append(dl)
            svs.append(sv)
            mis.append(mi)
            incls.append(plsc.cumsum(mi))
        o = off
        for u in range(4):
            m = mis[u] > 0
            pos = jnp.where(m, o + (incls[u] - mis[u]), TRASH + lanes)
            plsc.store_scatter(stg_s, [pos], svs[u])
            plsc.store_scatter(stg_d, [pos], dvs[u])
            o = o + incls[u][15]
        off = o

        def do_flush(o, h):
            hb = pl.multiple_of(lbase + h, 8)
            pltpu.sync_copy(stg_s.at[pl.ds(0, FLUSH)],
                            srcl_hbm.at[pl.ds(hb, FLUSH)])
            pltpu.sync_copy(stg_d.at[pl.ds(0, FLUSH)],
                            dstl_hbm.at[pl.ds(hb, FLUSH)])
            for u in range(4):
                stg_s[pl.ds(16 * u, 16)] = stg_s[pl.ds(FLUSH + 16 * u, 16)]
                stg_d[pl.ds(16 * u, 16)] = stg_d[pl.ds(FLUSH + 16 * u, 16)]
            return o - FLUSH, h + FLUSH

        return lax.cond(off >= FLUSH, do_flush, lambda o, h: (o, h), off, hoff)

    def chunk_body(ci, carry):
        pltpu.sync_copy(src_hbm.at[pl.ds(ci * FCH, FCH)], schunk)
        pltpu.sync_copy(dst_hbm.at[pl.ds(ci * FCH, FCH)], dchunk)
        return lax.fori_loop(0, FCH // 64, inner, carry)

    off, hoff = lax.fori_loop(0, E // FCH, chunk_body, (0, 0))

    a0 = (off // 16) * 16
    rem = off - a0
    keep_s = stg_s[pl.ds(a0, 16)]
    keep_d = stg_d[pl.ds(a0, 16)]
    stg_s[pl.ds(a0, 16)] = jnp.where(lanes < rem, keep_s, 0)
    stg_d[pl.ds(a0, 16)] = jnp.where(lanes < rem, keep_d, NPT)
    for k in range(1, PADM // 16):
        stg_s[pl.ds(a0 + 16 * k, 16)] = jnp.zeros((16,), jnp.int32)
        stg_d[pl.ds(a0 + 16 * k, 16)] = jnp.full((16,), NPT, jnp.int32)
    hb = pl.multiple_of(lbase + hoff, 8)
    pltpu.sync_copy(stg_s.at[pl.ds(0, FLUSH)], srcl_hbm.at[pl.ds(hb, FLUSH)])
    pltpu.sync_copy(stg_d.at[pl.ds(0, FLUSH)], dstl_hbm.at[pl.ds(hb, FLUSH)])

    total = hoff + ((off + PADM - 1) // PADM) * PADM
    cvec[...] = jnp.broadcast_to(total, (16,)).astype(jnp.int32)
    pltpu.sync_copy(cvec, cnt_hbm.at[pl.ds(pl.multiple_of(wid * 16, 8), 16)])


@functools.partial(
    pl.kernel,
    mesh=_mesh,
    compiler_params=_sc_params,
    out_type=jax.ShapeDtypeStruct((NPAD, H), jnp.float32),
    scratch_types=[
        pltpu.VMEM((NPT + 1, H), jnp.float32),
        pltpu.VMEM((2, CH), jnp.int32),
        pltpu.VMEM((2, CH, H), jnp.float32),
        pltpu.VMEM((2, CH), jnp.int32),
        pltpu.VMEM((16,), jnp.int32),
        pltpu.SemaphoreType.DMA,
        pltpu.SemaphoreType.DMA,
        pltpu.SemaphoreType.DMA,
        pltpu.SemaphoreType.DMA,
    ],
)
def _sc_segmax(q_hbm, srcl_hbm, dstl_hbm, cnt_hbm, s_hbm,
               s_loc, idx_v, rows_v, dl_vmem, cnt_vmem, sem0, sem1,
               isem0, isem1):
    wid = _wid()
    base = pl.multiple_of(wid * NPT, 8)
    lbase = pl.multiple_of(wid * ECAP, 8)
    pltpu.sync_copy(cnt_hbm.at[pl.ds(pl.multiple_of(wid * 16, 8), 16)], cnt_vmem)
    n = cnt_vmem[...][0]
    ng = n // CH

    negv = jnp.full((16,), NEG, jnp.float32)

    def initb(i, _):
        for c in range(H // 16):
            s_loc[i, pl.ds(c * 16, 16)] = negv
        return 0

    lax.fori_loop(0, NPT + 1, initb, 0)

    def prefetch_idx(slot, isem, g):
        gb = pl.multiple_of(lbase + g * CH, 8)
        pltpu.async_copy(srcl_hbm.at[pl.ds(gb, CH)], idx_v.at[slot], isem)

    def fire(slot, sem, isem, g):
        gb = pl.multiple_of(lbase + g * CH, 8)
        pltpu.make_async_copy(srcl_hbm.at[pl.ds(0, CH)], idx_v.at[slot],
                              isem).wait()
        pltpu.async_copy(q_hbm.at[idx_v.at[slot]], rows_v.at[slot], sem)
        pltpu.async_copy(dstl_hbm.at[pl.ds(gb, CH)], dl_vmem.at[slot], sem)

    def wait(slot, sem):
        pltpu.make_async_copy(q_hbm.at[pl.ds(0, CH)], rows_v.at[slot], sem).wait()
        pltpu.make_async_copy(dstl_hbm.at[pl.ds(0, CH)], dl_vmem.at[slot],
                              sem).wait()

    def drain(slot, g):
        def group_body(gi, _):
            dl16 = dl_vmem[slot, pl.ds(gi * 16, 16)]
            for t in range(16):
                d = dl16[t]
                e = gi * 16 + t
                for c in range(H // 16):
                    sl = pl.ds(c * 16, 16)
                    s_loc[d, sl] = jnp.maximum(s_loc[d, sl],
                                               rows_v[slot, e, sl])
            return 0

        lax.fori_loop(0, CH // 16, group_body, 0, unroll=2)

    @pl.when(ng > 0)
    def _():
        prefetch_idx(0, isem0, 0)
        prefetch_idx(1, isem1, 1)
        fire(0, sem0, isem0, 0)

        @pl.when(2 < ng)
        def _():
            prefetch_idx(0, isem0, 2)

    def body(i, _):
        g0 = 2 * i
        fire(1, sem1, isem1, g0 + 1)

        @pl.when(g0 + 3 < ng)
        def _():
            prefetch_idx(1, isem1, g0 + 3)

        wait(0, sem0)
        drain(0, g0)

        @pl.when(g0 + 2 < ng)
        def _():
            fire(0, sem0, isem0, g0 + 2)

        @pl.when(g0 + 4 < ng)
        def _():
            prefetch_idx(0, isem0, g0 + 4)

        wait(1, sem1)
        drain(1, g0 + 1)
        return 0

    lax.fori_loop(0, ng // 2, body, 0)
    pltpu.sync_copy(s_loc.at[pl.ds(0, NPT)], s_hbm.at[pl.ds(base, NPT)])


def _mm_in_body(x_ref, w_ref, p_ref, q_ref):
    pq = jnp.dot(x_ref[...], w_ref[...], preferred_element_type=jnp.float32, precision=lax.Precision.HIGHEST)
    p_ref[...] = pq[:, :H]
    q_ref[...] = pq[:, H:]


def _mm_in(xp, wc):
    return pl.pallas_call(
        _mm_in_body,
        grid=(4,),
        in_specs=[pl.BlockSpec((RB, F_IN), lambda i: (i, 0)),
                  pl.BlockSpec((F_IN, 2 * H), lambda i: (0, 0))],
        out_specs=[pl.BlockSpec((RB, H), lambda i: (i, 0)),
                   pl.BlockSpec((RB, H), lambda i: (i, 0))],
        out_shape=[jax.ShapeDtypeStruct((NPAD, H), jnp.float32)] * 2,
    )(xp, wc)


def _stats_body(p_ref, b_ref, s_ref, h_ref, sums_ref):
    i = pl.program_id(0)
    h = jnp.maximum(p_ref[...] + b_ref[...] + s_ref[...], 0.0)
    h_ref[...] = h
    ps = jnp.sum(h, 0, keepdims=True)

    @pl.when(i == 0)
    def _():
        sums_ref[...] = ps

    @pl.when(i > 0)
    def _():
        sums_ref[...] = sums_ref[...] + ps


def _stats(p, b, s):
    return pl.pallas_call(
        _stats_body,
        grid=(4,),
        in_specs=[pl.BlockSpec((RB, H), lambda i: (i, 0)),
                  pl.BlockSpec((1, H), lambda i: (0, 0)),
                  pl.BlockSpec((RB, H), lambda i: (i, 0))],
        out_specs=[pl.BlockSpec((RB, H), lambda i: (i, 0)),
                   pl.BlockSpec((1, H), lambda i: (0, 0))],
        out_shape=[jax.ShapeDtypeStruct((NPAD, H), jnp.float32),
                   jax.ShapeDtypeStruct((1, H), jnp.float32)],
    )(p, b, s)


def _var_body(h_ref, sum_ref, var_ref):
    i = pl.program_id(0)
    mu = sum_ref[...] * (1.0 / N)
    d = h_ref[...] - mu
    npad_rows = NPAD - N
    ps = jnp.sum(d * d, 0, keepdims=True)

    @pl.when(i == 0)
    def _():
        var_ref[...] = ps

    @pl.when(i > 0)
    def _():
        var_ref[...] = var_ref[...] + ps

    @pl.when(i == 3)
    def _():
        var_ref[...] = (var_ref[...] - npad_rows * mu * mu) * (1.0 / N)


def _var(h, s):
    return pl.pallas_call(
        _var_body,
        grid=(4,),
        in_specs=[pl.BlockSpec((RB, H), lambda i: (i, 0)),
                  pl.BlockSpec((1, H), lambda i: (0, 0))],
        out_specs=pl.BlockSpec((1, H), lambda i: (0, 0)),
        out_shape=jax.ShapeDtypeStruct((1, H), jnp.float32),
    )(h, s)


def _bn_from_sums(sum_ref, var_ref, g_ref):
    mu = sum_ref[...] * (1.0 / N)
    scale = g_ref[...] * lax.rsqrt(var_ref[...] + 1e-5)
    return mu, scale


def _bnmm_body(h_ref, sum_ref, var_ref, g_ref, be_ref, w_ref, p_ref, q_ref):
    mu, scale = _bn_from_sums(sum_ref, var_ref, g_ref)
    hn = jnp.maximum((h_ref[...] - mu) * scale + be_ref[...], 0.0)
    pq = jnp.dot(hn, w_ref[...], preferred_element_type=jnp.float32, precision=lax.Precision.HIGHEST)
    p_ref[...] = pq[:, :H]
    q_ref[...] = pq[:, H:]


def _bnmm(h, s, v, g, be, wc):
    return pl.pallas_call(
        _bnmm_body,
        grid=(4,),
        in_specs=[pl.BlockSpec((RB, H), lambda i: (i, 0)),
                  pl.BlockSpec((1, H), lambda i: (0, 0)),
                  pl.BlockSpec((1, H), lambda i: (0, 0)),
                  pl.BlockSpec((1, H), lambda i: (0, 0)),
                  pl.BlockSpec((1, H), lambda i: (0, 0)),
                  pl.BlockSpec((H, 2 * H), lambda i: (0, 0))],
        out_specs=[pl.BlockSpec((RB, H), lambda i: (i, 0)),
                   pl.BlockSpec((RB, H), lambda i: (i, 0))],
        out_shape=[jax.ShapeDtypeStruct((NPAD, H), jnp.float32)] * 2,
    )(h, s, v, g, be, wc)


def _final_body(h_ref, sum_ref, var_ref, g_ref, be_ref, batch_ref, wl_ref,
                bl_ref, out_ref):
    mu, scale = _bn_from_sums(sum_ref, var_ref, g_ref)
    hn = (h_ref[...] - mu) * scale + be_ref[...]
    gids = lax.broadcasted_iota(jnp.int32, (G, NPAD), 0)
    m = (batch_ref[...] == gids).astype(jnp.float32)
    sums_g = jnp.dot(m, hn, preferred_element_type=jnp.float32, precision=lax.Precision.HIGHEST)
    counts = jnp.sum(m, axis=1, keepdims=True)
    pooled = sums_g / jnp.maximum(counts, 1.0)
    out = jnp.dot(pooled, wl_ref[...], preferred_element_type=jnp.float32, precision=lax.Precision.HIGHEST)
    out_ref[...] = jnp.maximum(out + bl_ref[...], 0.0)


def _final(h, s, v, g, be, batch2d, wlp, blv):
    return pl.pallas_call(
        _final_body,
        grid=(1,),
        in_specs=[pl.BlockSpec((NPAD, H), lambda i: (0, 0)),
                  pl.BlockSpec((1, H), lambda i: (0, 0)),
                  pl.BlockSpec((1, H), lambda i: (0, 0)),
                  pl.BlockSpec((1, H), lambda i: (0, 0)),
                  pl.BlockSpec((1, H), lambda i: (0, 0)),
                  pl.BlockSpec((1, NPAD), lambda i: (0, 0)),
                  pl.BlockSpec((H, H), lambda i: (0, 0)),
                  pl.BlockSpec((1, H), lambda i: (0, 0))],
        out_specs=pl.BlockSpec((G, H), lambda i: (0, 0)),
        out_shape=jax.ShapeDtypeStruct((G, H), jnp.float32),
    )(h, s, v, g, be, batch2d, wlp, blv)


def kernel(x, edge_index, batch, W1, b1, W2, b2, W3, b3,
           g1, be1, g2, be2, g3, be3, Wl, bl):
    x = x.astype(jnp.float32).reshape(-1, F_IN)
    src = edge_index[0].astype(jnp.int32)
    dst = edge_index[1].astype(jnp.int32)
    batch2d = jnp.pad(batch.astype(jnp.int32), (0, NPAD - N),
                      constant_values=G).reshape(1, NPAD)
    xp = jnp.pad(x, ((0, NPAD - N), (0, 0)))

    def split(w):
        f = w.shape[0] // 2
        return jnp.concatenate([w[:f] - w[f:], w[f:]], axis=1)

    wc1, wc2, wc3 = split(W1), split(W2), split(W3)
    b1r, b2r, b3r = (v.reshape(1, H) for v in (b1, b2, b3))
    g1r, g2r, g3r = (v.reshape(1, H) for v in (g1, g2, g3))
    be1r, be2r, be3r = (v.reshape(1, H) for v in (be1, be2, be3))
    wlp = jnp.zeros((H, H), jnp.float32).at[:, 0].set(Wl[:, 0])
    blv = jnp.zeros((1, H), jnp.float32).at[0, 0].set(bl[0])

    srcl, dstl, cnts = _sc_filter(src, dst)

    p1, q1 = _mm_in(xp, wc1)
    s1 = _sc_segmax(q1, srcl, dstl, cnts)
    h1, sum1 = _stats(p1, b1r, s1)
    var1 = _var(h1, sum1)

    p2, q2 = _bnmm(h1, sum1, var1, g1r, be1r, wc2)
    s2 = _sc_segmax(q2, srcl, dstl, cnts)
    h2, sum2 = _stats(p2, b2r, s2)
    var2 = _var(h2, sum2)

    p3, q3 = _bnmm(h2, sum2, var2, g2r, be2r, wc3)
    s3 = _sc_segmax(q3, srcl, dstl, cnts)
    h3, sum3 = _stats(p3, b3r, s3)
    var3 = _var(h3, sum3)

    out = _final(h3, sum3, var3, g3r, be3r, batch2d, wlp, blv)
    return out[:, :1]


# speedup vs baseline: 1.4572x; 1.0249x over previous
import functools

import jax
import jax.numpy as jnp
from jax import lax
from jax.experimental import pallas as pl
from jax.experimental.pallas import tpu as pltpu
from jax.experimental.pallas import tpu_sc as plsc

N = 10000
E = 320000
F_IN = 24
H = 128
G = 64

NT = 32
NPT = 320
NPAD = NT * NPT
RB = NPAD // 4

CH = 128
PADM = 2 * CH
FLUSH = 2048
FCH = 8000
STG = FLUSH + PADM + 128
TRASH = STG - 16
ECAP = E + 2 * FLUSH
NEG = -1.0e30

_mesh = plsc.VectorSubcoreMesh(core_axis_name="c", subcore_axis_name="s")
_sc_params = pltpu.CompilerParams(needs_layout_passes=False)


def _wid():
    return lax.axis_index("s") * 2 + lax.axis_index("c")


@functools.partial(
    pl.kernel,
    mesh=_mesh,
    compiler_params=_sc_params,
    out_type=[
        jax.ShapeDtypeStruct((NT * ECAP,), jnp.int32),
        jax.ShapeDtypeStruct((NT * ECAP,), jnp.int32),
        jax.ShapeDtypeStruct((NT * 16,), jnp.int32),
    ],
    scratch_types=[
        pltpu.VMEM((FCH,), jnp.int32),
        pltpu.VMEM((FCH,), jnp.int32),
        pltpu.VMEM((STG,), jnp.int32),
        pltpu.VMEM((STG,), jnp.int32),
        pltpu.VMEM((16,), jnp.int32),
    ],
)
def _sc_filter(src_hbm, dst_hbm, srcl_hbm, dstl_hbm, cnt_hbm,
               dchunk, schunk, stg_s, stg_d, cvec):
    wid = _wid()
    base = wid * NPT
    lbase = pl.multiple_of(wid * ECAP, 8)
    lanes = lax.iota(jnp.int32, 16)

    def inner(j, carry):
        off, hoff = carry
        dvs, svs, mis, incls = [], [], [], []
        for u in range(4):
            dv = dchunk[pl.ds(j * 64 + 16 * u, 16)]
            sv = schunk[pl.ds(j * 64 + 16 * u, 16)]
            dl = dv - base
            m = (dl >= 0) & (dl < NPT)
            mi = jnp.where(m, 1, 0)
            dvs.append(dl)
            svs.append(sv)
            mis.append(mi)
            incls.append(plsc.cumsum(mi))
        o = off
        for u in range(4):
            m = mis[u] > 0
            pos = jnp.where(m, o + (incls[u] - mis[u]), TRASH + lanes)
            plsc.store_scatter(stg_s, [pos], svs[u])
            plsc.store_scatter(stg_d, [pos], dvs[u])
            o = o + incls[u][15]
        off = o

        def do_flush(o, h):
            hb = pl.multiple_of(lbase + h, 8)
            pltpu.sync_copy(stg_s.at[pl.ds(0, FLUSH)],
                            srcl_hbm.at[pl.ds(hb, FLUSH)])
            pltpu.sync_copy(stg_d.at[pl.ds(0, FLUSH)],
                            dstl_hbm.at[pl.ds(hb, FLUSH)])
            for u in range(4):
                stg_s[pl.ds(16 * u, 16)] = stg_s[pl.ds(FLUSH + 16 * u, 16)]
                stg_d[pl.ds(16 * u, 16)] = stg_d[pl.ds(FLUSH + 16 * u, 16)]
            return o - FLUSH, h + FLUSH

        return lax.cond(off >= FLUSH, do_flush, lambda o, h: (o, h), off, hoff)

    def chunk_body(ci, carry):
        pltpu.sync_copy(src_hbm.at[pl.ds(ci * FCH, FCH)], schunk)
        pltpu.sync_copy(dst_hbm.at[pl.ds(ci * FCH, FCH)], dchunk)
        return lax.fori_loop(0, FCH // 64, inner, carry)

    off, hoff = lax.fori_loop(0, E // FCH, chunk_body, (0, 0))

    a0 = (off // 16) * 16
    rem = off - a0
    keep_s = stg_s[pl.ds(a0, 16)]
    keep_d = stg_d[pl.ds(a0, 16)]
    stg_s[pl.ds(a0, 16)] = jnp.where(lanes < rem, keep_s, 0)
    stg_d[pl.ds(a0, 16)] = jnp.where(lanes < rem, keep_d, NPT)
    for k in range(1, PADM // 16):
        stg_s[pl.ds(a0 + 16 * k, 16)] = jnp.zeros((16,), jnp.int32)
        stg_d[pl.ds(a0 + 16 * k, 16)] = jnp.full((16,), NPT, jnp.int32)
    hb = pl.multiple_of(lbase + hoff, 8)
    pltpu.sync_copy(stg_s.at[pl.ds(0, FLUSH)], srcl_hbm.at[pl.ds(hb, FLUSH)])
    pltpu.sync_copy(stg_d.at[pl.ds(0, FLUSH)], dstl_hbm.at[pl.ds(hb, FLUSH)])

    total = hoff + ((off + PADM - 1) // PADM) * PADM
    cvec[...] = jnp.broadcast_to(total, (16,)).astype(jnp.int32)
    pltpu.sync_copy(cvec, cnt_hbm.at[pl.ds(pl.multiple_of(wid * 16, 8), 16)])


@functools.partial(
    pl.kernel,
    mesh=_mesh,
    compiler_params=_sc_params,
    out_type=jax.ShapeDtypeStruct((NPAD, H), jnp.float32),
    scratch_types=[
        pltpu.VMEM((NPT + 1, H), jnp.float32),
        pltpu.VMEM((2, CH), jnp.int32),
        pltpu.VMEM((2, CH, H), jnp.float32),
        pltpu.VMEM((2, CH), jnp.int32),
        pltpu.VMEM((16,), jnp.int32),
        pltpu.SemaphoreType.DMA,
        pltpu.SemaphoreType.DMA,
        pltpu.SemaphoreType.DMA,
        pltpu.SemaphoreType.DMA,
    ],
)
def _sc_segmax(q_hbm, srcl_hbm, dstl_hbm, cnt_hbm, s_hbm,
               s_loc, idx_v, rows_v, dl_vmem, cnt_vmem, sem0, sem1,
               isem0, isem1):
    wid = _wid()
    base = pl.multiple_of(wid * NPT, 8)
    lbase = pl.multiple_of(wid * ECAP, 8)
    pltpu.sync_copy(cnt_hbm.at[pl.ds(pl.multiple_of(wid * 16, 8), 16)], cnt_vmem)
    n = cnt_vmem[...][0]
    ng = n // CH

    negv = jnp.full((16,), NEG, jnp.float32)

    def initb(i, _):
        for c in range(H // 16):
            s_loc[i, pl.ds(c * 16, 16)] = negv
        return 0

    lax.fori_loop(0, NPT + 1, initb, 0)

    def prefetch_idx(slot, isem, g):
        gb = pl.multiple_of(lbase + g * CH, 8)
        pltpu.async_copy(srcl_hbm.at[pl.ds(gb, CH)], idx_v.at[slot], isem)

    def fire(slot, sem, isem, g):
        gb = pl.multiple_of(lbase + g * CH, 8)
        pltpu.make_async_copy(srcl_hbm.at[pl.ds(0, CH)], idx_v.at[slot],
                              isem).wait()
        pltpu.async_copy(q_hbm.at[idx_v.at[slot]], rows_v.at[slot], sem)
        pltpu.async_copy(dstl_hbm.at[pl.ds(gb, CH)], dl_vmem.at[slot], sem)

    def wait(slot, sem):
        pltpu.make_async_copy(q_hbm.at[pl.ds(0, CH)], rows_v.at[slot], sem).wait()
        pltpu.make_async_copy(dstl_hbm.at[pl.ds(0, CH)], dl_vmem.at[slot],
                              sem).wait()

    def drain(slot, g):
        def group_body(gi, _):
            dl16 = dl_vmem[slot, pl.ds(gi * 16, 16)]
            for t in range(16):
                d = dl16[t]
                e = gi * 16 + t
                for c in range(H // 16):
                    sl = pl.ds(c * 16, 16)
                    s_loc[d, sl] = jnp.maximum(s_loc[d, sl],
                                               rows_v[slot, e, sl])
            return 0

        lax.fori_loop(0, CH // 16, group_body, 0, unroll=2)

    @pl.when(ng > 0)
    def _():
        prefetch_idx(0, isem0, 0)
        fire(0, sem0, isem0, 0)
        prefetch_idx(1, isem1, 1)

    def body(i, _):
        g0 = 2 * i
        fire(1, sem1, isem1, g0 + 1)
        wait(0, sem0)

        @pl.when(g0 + 2 < ng)
        def _():
            prefetch_idx(0, isem0, g0 + 2)

        drain(0, g0)

        @pl.when(g0 + 2 < ng)
        def _():
            fire(0, sem0, isem0, g0 + 2)

        wait(1, sem1)

        @pl.when(g0 + 3 < ng)
        def _():
            prefetch_idx(1, isem1, g0 + 3)

        drain(1, g0 + 1)
        return 0

    lax.fori_loop(0, ng // 2, body, 0)
    pltpu.sync_copy(s_loc.at[pl.ds(0, NPT)], s_hbm.at[pl.ds(base, NPT)])


def _mm_in_body(x_ref, w_ref, p_ref, q_ref):
    pq = jnp.dot(x_ref[...], w_ref[...], preferred_element_type=jnp.float32, precision=lax.Precision.HIGHEST)
    p_ref[...] = pq[:, :H]
    q_ref[...] = pq[:, H:]


def _mm_in(xp, wc):
    return pl.pallas_call(
        _mm_in_body,
        grid=(4,),
        in_specs=[pl.BlockSpec((RB, F_IN), lambda i: (i, 0)),
                  pl.BlockSpec((F_IN, 2 * H), lambda i: (0, 0))],
        out_specs=[pl.BlockSpec((RB, H), lambda i: (i, 0)),
                   pl.BlockSpec((RB, H), lambda i: (i, 0))],
        out_shape=[jax.ShapeDtypeStruct((NPAD, H), jnp.float32)] * 2,
    )(xp, wc)


def _stats_body(p_ref, b_ref, s_ref, h_ref, sums_ref):
    i = pl.program_id(0)
    h = jnp.maximum(p_ref[...] + b_ref[...] + s_ref[...], 0.0)
    h_ref[...] = h
    ps = jnp.sum(h, 0, keepdims=True)

    @pl.when(i == 0)
    def _():
        sums_ref[...] = ps

    @pl.when(i > 0)
    def _():
        sums_ref[...] = sums_ref[...] + ps


def _stats(p, b, s):
    return pl.pallas_call(
        _stats_body,
        grid=(4,),
        in_specs=[pl.BlockSpec((RB, H), lambda i: (i, 0)),
                  pl.BlockSpec((1, H), lambda i: (0, 0)),
                  pl.BlockSpec((RB, H), lambda i: (i, 0))],
        out_specs=[pl.BlockSpec((RB, H), lambda i: (i, 0)),
                   pl.BlockSpec((1, H), lambda i: (0, 0))],
        out_shape=[jax.ShapeDtypeStruct((NPAD, H), jnp.float32),
                   jax.ShapeDtypeStruct((1, H), jnp.float32)],
    )(p, b, s)


def _var_body(h_ref, sum_ref, var_ref):
    i = pl.program_id(0)
    mu = sum_ref[...] * (1.0 / N)
    d = h_ref[...] - mu
    npad_rows = NPAD - N
    ps = jnp.sum(d * d, 0, keepdims=True)

    @pl.when(i == 0)
    def _():
        var_ref[...] = ps

    @pl.when(i > 0)
    def _():
        var_ref[...] = var_ref[...] + ps

    @pl.when(i == 3)
    def _():
        var_ref[...] = (var_ref[...] - npad_rows * mu * mu) * (1.0 / N)


def _var(h, s):
    return pl.pallas_call(
        _var_body,
        grid=(4,),
        in_specs=[pl.BlockSpec((RB, H), lambda i: (i, 0)),
                  pl.BlockSpec((1, H), lambda i: (0, 0))],
        out_specs=pl.BlockSpec((1, H), lambda i: (0, 0)),
        out_shape=jax.ShapeDtypeStruct((1, H), jnp.float32),
    )(h, s)


def _bn_from_sums(sum_ref, var_ref, g_ref):
    mu = sum_ref[...] * (1.0 / N)
    scale = g_ref[...] * lax.rsqrt(var_ref[...] + 1e-5)
    return mu, scale


def _bnmm_body(h_ref, sum_ref, var_ref, g_ref, be_ref, w_ref, p_ref, q_ref):
    mu, scale = _bn_from_sums(sum_ref, var_ref, g_ref)
    hn = jnp.maximum((h_ref[...] - mu) * scale + be_ref[...], 0.0)
    pq = jnp.dot(hn, w_ref[...], preferred_element_type=jnp.float32, precision=lax.Precision.HIGHEST)
    p_ref[...] = pq[:, :H]
    q_ref[...] = pq[:, H:]


def _bnmm(h, s, v, g, be, wc):
    return pl.pallas_call(
        _bnmm_body,
        grid=(4,),
        in_specs=[pl.BlockSpec((RB, H), lambda i: (i, 0)),
                  pl.BlockSpec((1, H), lambda i: (0, 0)),
                  pl.BlockSpec((1, H), lambda i: (0, 0)),
                  pl.BlockSpec((1, H), lambda i: (0, 0)),
                  pl.BlockSpec((1, H), lambda i: (0, 0)),
                  pl.BlockSpec((H, 2 * H), lambda i: (0, 0))],
        out_specs=[pl.BlockSpec((RB, H), lambda i: (i, 0)),
                   pl.BlockSpec((RB, H), lambda i: (i, 0))],
        out_shape=[jax.ShapeDtypeStruct((NPAD, H), jnp.float32)] * 2,
    )(h, s, v, g, be, wc)


def _final_body(h_ref, sum_ref, var_ref, g_ref, be_ref, batch_ref, wl_ref,
                bl_ref, out_ref):
    mu, scale = _bn_from_sums(sum_ref, var_ref, g_ref)
    hn = (h_ref[...] - mu) * scale + be_ref[...]
    gids = lax.broadcasted_iota(jnp.int32, (G, NPAD), 0)
    m = (batch_ref[...] == gids).astype(jnp.float32)
    sums_g = jnp.dot(m, hn, preferred_element_type=jnp.float32, precision=lax.Precision.HIGHEST)
    counts = jnp.sum(m, axis=1, keepdims=True)
    pooled = sums_g / jnp.maximum(counts, 1.0)
    out = jnp.dot(pooled, wl_ref[...], preferred_element_type=jnp.float32, precision=lax.Precision.HIGHEST)
    out_ref[...] = jnp.maximum(out + bl_ref[...], 0.0)


def _final(h, s, v, g, be, batch2d, wlp, blv):
    return pl.pallas_call(
        _final_body,
        grid=(1,),
        in_specs=[pl.BlockSpec((NPAD, H), lambda i: (0, 0)),
                  pl.BlockSpec((1, H), lambda i: (0, 0)),
                  pl.BlockSpec((1, H), lambda i: (0, 0)),
                  pl.BlockSpec((1, H), lambda i: (0, 0)),
                  pl.BlockSpec((1, H), lambda i: (0, 0)),
                  pl.BlockSpec((1, NPAD), lambda i: (0, 0)),
                  pl.BlockSpec((H, H), lambda i: (0, 0)),
                  pl.BlockSpec((1, H), lambda i: (0, 0))],
        out_specs=pl.BlockSpec((G, H), lambda i: (0, 0)),
        out_shape=jax.ShapeDtypeStruct((G, H), jnp.float32),
    )(h, s, v, g, be, batch2d, wlp, blv)


def kernel(x, edge_index, batch, W1, b1, W2, b2, W3, b3,
           g1, be1, g2, be2, g3, be3, Wl, bl):
    x = x.astype(jnp.float32).reshape(-1, F_IN)
    src = edge_index[0].astype(jnp.int32)
    dst = edge_index[1].astype(jnp.int32)
    batch2d = jnp.pad(batch.astype(jnp.int32), (0, NPAD - N),
                      constant_values=G).reshape(1, NPAD)
    xp = jnp.pad(x, ((0, NPAD - N), (0, 0)))

    def split(w):
        f = w.shape[0] // 2
        return jnp.concatenate([w[:f] - w[f:], w[f:]], axis=1)

    wc1, wc2, wc3 = split(W1), split(W2), split(W3)
    b1r, b2r, b3r = (v.reshape(1, H) for v in (b1, b2, b3))
    g1r, g2r, g3r = (v.reshape(1, H) for v in (g1, g2, g3))
    be1r, be2r, be3r = (v.reshape(1, H) for v in (be1, be2, be3))
    wlp = jnp.zeros((H, H), jnp.float32).at[:, 0].set(Wl[:, 0])
    blv = jnp.zeros((1, H), jnp.float32).at[0, 0].set(bl[0])

    srcl, dstl, cnts = _sc_filter(src, dst)

    p1, q1 = _mm_in(xp, wc1)
    s1 = _sc_segmax(q1, srcl, dstl, cnts)
    h1, sum1 = _stats(p1, b1r, s1)
    var1 = _var(h1, sum1)

    p2, q2 = _bnmm(h1, sum1, var1, g1r, be1r, wc2)
    s2 = _sc_segmax(q2, srcl, dstl, cnts)
    h2, sum2 = _stats(p2, b2r, s2)
    var2 = _var(h2, sum2)

    p3, q3 = _bnmm(h2, sum2, var2, g2r, be2r, wc3)
    s3 = _sc_segmax(q3, srcl, dstl, cnts)
    h3, sum3 = _stats(p3, b3r, s3)
    var3 = _var(h3, sum3)

    out = _final(h3, sum3, var3, g3r, be3r, batch2d, wlp, blv)
    return out[:, :1]
